# baseline (speedup 1.0000x reference)
import functools

import jax
import jax.numpy as jnp
from jax.experimental import pallas as pl
from jax.experimental.pallas import tpu as pltpu

LAMBDA_REGRESSION = 2.0

B, C, H, W = 16, 19, 128, 128
N = H * W
NB = 4096
NJ = N // NB


def _body(xi_ref, ti_ref, xr1_ref, tr1_ref, xr2_ref, tr2_ref,
          out_ref, acc_ref):
    b = pl.program_id(0)
    j = pl.program_id(1)

    @pl.when(jnp.logical_and(b == 0, j == 0))
    def _init():
        for k in range(5):
            acc_ref[k] = 0.0

    ti = ti_ref[0]
    tgt = ti[:C]
    mask = (jnp.sum(ti, axis=0, keepdims=True) > 0.0).astype(jnp.float32)

    xi = xi_ref[0]
    log_x = jnp.maximum(jnp.log(xi), -100.0)
    log_1mx = jnp.maximum(jnp.log(1.0 - xi), -100.0)
    bce = -(tgt * log_x + (1.0 - tgt) * log_1mx)
    acc_ref[0] += jnp.sum(mask * bce)
    acc_ref[1] += jnp.sum(mask)

    rmask = (tgt > 0.0).astype(jnp.float32)
    acc_ref[2] += jnp.sum(rmask)
    rm3 = rmask[:, None, :]
    d1 = jnp.abs(xr1_ref[0] - tr1_ref[0])
    acc_ref[3] += jnp.sum(rm3 * d1)
    d2 = jnp.abs(xr2_ref[0] - tr2_ref[0])
    acc_ref[4] += jnp.sum(rm3 * d2)

    @pl.when(jnp.logical_and(b == B - 1, j == NJ - 1))
    def _finish():
        n_sel = jnp.float32(C) * acc_ref[1]
        n_reg = 2.0 * acc_ref[2]
        out_ref[0] = acc_ref[0] / n_sel
        scale = LAMBDA_REGRESSION / 1000.0 / jnp.float32(B)
        out_ref[1] = scale * acc_ref[3] / n_reg
        out_ref[2] = scale * acc_ref[4] / n_reg


@functools.partial(jax.jit, static_argnames=("interpret",))
def kernel(x_intensity, x_reg1, x_reg2, target_intensity, target_reg1,
           target_reg2, target_scale, interpret=False):
    del target_scale
    xi = x_intensity.reshape(B, C, N)
    ti = target_intensity.reshape(B, C + 1, N)

    spec3 = lambda c: pl.BlockSpec((1, c, NB), lambda b, j: (b, 0, j))
    spec4 = pl.BlockSpec((1, C, 2, NB), lambda b, j: (b, 0, 0, j))

    out = pl.pallas_call(
        _body,
        grid=(B, NJ),
        in_specs=[spec3(C), spec3(C + 1), spec4, spec4, spec4, spec4],
        out_specs=pl.BlockSpec(memory_space=pltpu.MemorySpace.SMEM),
        out_shape=jax.ShapeDtypeStruct((3,), jnp.float32),
        scratch_shapes=[pltpu.SMEM((5,), jnp.float32)],
        interpret=interpret,
    )(xi, ti, x_reg1.reshape(B, C, 2, N), target_reg1.reshape(B, C, 2, N),
      x_reg2.reshape(B, C, 2, N), target_reg2.reshape(B, C, 2, N))
    return (out[0], out[1], out[2])


# speedup vs baseline: 4.0756x; 4.0756x over previous
import functools

import jax
import jax.numpy as jnp
from jax.experimental import pallas as pl
from jax.experimental.pallas import tpu as pltpu

LAMBDA_REGRESSION = 2.0

B, C, H, W = 16, 19, 128, 128
HB = 32
NJ = H // HB


def _body(xi_ref, ti_ref, xr1_ref, tr1_ref, xr2_ref, tr2_ref,
          out_ref, acc_ref):
    b = pl.program_id(0)
    j = pl.program_id(1)

    @pl.when(jnp.logical_and(b == 0, j == 0))
    def _init():
        for k in range(5):
            acc_ref[k] = 0.0

    ti = ti_ref[...]
    tgt = ti[:, :C]
    mask = (jnp.sum(ti, axis=1, keepdims=True) > 0.0).astype(jnp.float32)

    xi = xi_ref[...]
    log_x = jnp.maximum(jnp.log(xi), -100.0)
    log_1mx = jnp.maximum(jnp.log(1.0 - xi), -100.0)
    bce = -(tgt * log_x + (1.0 - tgt) * log_1mx)
    acc_ref[0] += jnp.sum(mask * bce)
    acc_ref[1] += jnp.sum(mask)

    rmask = (tgt > 0.0).astype(jnp.float32)
    acc_ref[2] += jnp.sum(rmask)
    d1 = jnp.abs(xr1_ref[...] - tr1_ref[...])
    acc_ref[3] += jnp.sum(rmask * d1)
    d2 = jnp.abs(xr2_ref[...] - tr2_ref[...])
    acc_ref[4] += jnp.sum(rmask * d2)

    @pl.when(jnp.logical_and(b == B - 1, j == NJ - 1))
    def _finish():
        n_sel = jnp.float32(C) * acc_ref[1]
        n_reg = 2.0 * acc_ref[2]
        out_ref[0] = acc_ref[0] / n_sel
        scale = LAMBDA_REGRESSION / 1000.0 / jnp.float32(B)
        out_ref[1] = scale * acc_ref[3] / n_reg
        out_ref[2] = scale * acc_ref[4] / n_reg


@functools.partial(jax.jit, static_argnames=("interpret",))
def kernel(x_intensity, x_reg1, x_reg2, target_intensity, target_reg1,
           target_reg2, target_scale, interpret=False):
    del target_scale

    spec1 = lambda c: pl.BlockSpec((1, c, 1, HB, W), lambda b, j: (b, 0, 0, j, 0))
    spec2 = pl.BlockSpec((1, C, 2, HB, W), lambda b, j: (b, 0, 0, j, 0))

    out = pl.pallas_call(
        _body,
        grid=(B, NJ),
        in_specs=[spec1(C), spec1(C + 1), spec2, spec2, spec2, spec2],
        out_specs=pl.BlockSpec(memory_space=pltpu.MemorySpace.SMEM),
        out_shape=jax.ShapeDtypeStruct((3,), jnp.float32),
        scratch_shapes=[pltpu.SMEM((5,), jnp.float32)],
        interpret=interpret,
    )(x_intensity, target_intensity, x_reg1, target_reg1, x_reg2, target_reg2)
    return (out[0], out[1], out[2])


# speedup vs baseline: 5.1576x; 1.2655x over previous
import functools

import jax
import jax.numpy as jnp
from jax.experimental import pallas as pl
from jax.experimental.pallas import tpu as pltpu

LAMBDA_REGRESSION = 2.0

B, C, H, W = 16, 19, 128, 128
HB = 64
NJ = H // HB


def _body(xi_ref, ti_ref, xr1_ref, tr1_ref, xr2_ref, tr2_ref,
          out_ref, acc_ref):
    b = pl.program_id(0)
    j = pl.program_id(1)

    @pl.when(jnp.logical_and(b == 0, j == 0))
    def _init():
        for k in range(5):
            acc_ref[k] = 0.0

    ti = ti_ref[...]
    tgt = ti[:, :C]
    mask = (jnp.sum(ti, axis=1, keepdims=True) > 0.0).astype(jnp.float32)

    xi = xi_ref[...]
    log_x = jnp.maximum(jnp.log(xi), -100.0)
    log_1mx = jnp.maximum(jnp.log(1.0 - xi), -100.0)
    bce = -(tgt * log_x + (1.0 - tgt) * log_1mx)
    acc_ref[0] += jnp.sum(mask * bce)
    acc_ref[1] += jnp.sum(mask)

    rmask = (tgt > 0.0).astype(jnp.float32)
    acc_ref[2] += jnp.sum(rmask)
    d1 = jnp.abs(xr1_ref[...] - tr1_ref[...])
    acc_ref[3] += jnp.sum(rmask * d1)
    d2 = jnp.abs(xr2_ref[...] - tr2_ref[...])
    acc_ref[4] += jnp.sum(rmask * d2)

    @pl.when(jnp.logical_and(b == B - 1, j == NJ - 1))
    def _finish():
        n_sel = jnp.float32(C) * acc_ref[1]
        n_reg = 2.0 * acc_ref[2]
        out_ref[0] = acc_ref[0] / n_sel
        scale = LAMBDA_REGRESSION / 1000.0 / jnp.float32(B)
        out_ref[1] = scale * acc_ref[3] / n_reg
        out_ref[2] = scale * acc_ref[4] / n_reg


@functools.partial(jax.jit, static_argnames=("interpret",))
def kernel(x_intensity, x_reg1, x_reg2, target_intensity, target_reg1,
           target_reg2, target_scale, interpret=False):
    del target_scale

    spec1 = lambda c: pl.BlockSpec((1, c, 1, HB, W), lambda b, j: (b, 0, 0, j, 0))
    spec2 = pl.BlockSpec((1, C, 2, HB, W), lambda b, j: (b, 0, 0, j, 0))

    out = pl.pallas_call(
        _body,
        grid=(B, NJ),
        in_specs=[spec1(C), spec1(C + 1), spec2, spec2, spec2, spec2],
        out_specs=pl.BlockSpec(memory_space=pltpu.MemorySpace.SMEM),
        out_shape=jax.ShapeDtypeStruct((3,), jnp.float32),
        scratch_shapes=[pltpu.SMEM((5,), jnp.float32)],
        interpret=interpret,
    )(x_intensity, target_intensity, x_reg1, target_reg1, x_reg2, target_reg2)
    return (out[0], out[1], out[2])


# speedup vs baseline: 5.5876x; 1.0834x over previous
import functools

import jax
import jax.numpy as jnp
from jax.experimental import pallas as pl
from jax.experimental.pallas import tpu as pltpu

LAMBDA_REGRESSION = 2.0

B, C, H, W = 16, 19, 128, 128
HB = 128
NJ = H // HB


def _body(xi_ref, ti_ref, xr1_ref, tr1_ref, xr2_ref, tr2_ref,
          out_ref, acc_ref):
    b = pl.program_id(0)
    j = pl.program_id(1)

    @pl.when(jnp.logical_and(b == 0, j == 0))
    def _init():
        for k in range(5):
            acc_ref[k] = 0.0

    ti = ti_ref[...]
    tgt = ti[:, :C]
    mask = (jnp.sum(ti, axis=1, keepdims=True) > 0.0).astype(jnp.float32)

    xi = xi_ref[...]
    log_x = jnp.maximum(jnp.log(xi), -100.0)
    log_1mx = jnp.maximum(jnp.log(1.0 - xi), -100.0)
    bce = -(tgt * log_x + (1.0 - tgt) * log_1mx)
    acc_ref[0] += jnp.sum(mask * bce)
    acc_ref[1] += jnp.sum(mask)

    rmask = (tgt > 0.0).astype(jnp.float32)
    acc_ref[2] += jnp.sum(rmask)
    d1 = jnp.abs(xr1_ref[...] - tr1_ref[...])
    acc_ref[3] += jnp.sum(rmask * d1)
    d2 = jnp.abs(xr2_ref[...] - tr2_ref[...])
    acc_ref[4] += jnp.sum(rmask * d2)

    @pl.when(jnp.logical_and(b == B - 1, j == NJ - 1))
    def _finish():
        n_sel = jnp.float32(C) * acc_ref[1]
        n_reg = 2.0 * acc_ref[2]
        out_ref[0] = acc_ref[0] / n_sel
        scale = LAMBDA_REGRESSION / 1000.0 / jnp.float32(B)
        out_ref[1] = scale * acc_ref[3] / n_reg
        out_ref[2] = scale * acc_ref[4] / n_reg


@functools.partial(jax.jit, static_argnames=("interpret",))
def kernel(x_intensity, x_reg1, x_reg2, target_intensity, target_reg1,
           target_reg2, target_scale, interpret=False):
    del target_scale

    spec1 = lambda c: pl.BlockSpec((1, c, 1, HB, W), lambda b, j: (b, 0, 0, j, 0))
    spec2 = pl.BlockSpec((1, C, 2, HB, W), lambda b, j: (b, 0, 0, j, 0))

    out = pl.pallas_call(
        _body,
        grid=(B, NJ),
        in_specs=[spec1(C), spec1(C + 1), spec2, spec2, spec2, spec2],
        out_specs=pl.BlockSpec(memory_space=pltpu.MemorySpace.SMEM),
        out_shape=jax.ShapeDtypeStruct((3,), jnp.float32),
        scratch_shapes=[pltpu.SMEM((5,), jnp.float32)],
        interpret=interpret,
    )(x_intensity, target_intensity, x_reg1, target_reg1, x_reg2, target_reg2)
    return (out[0], out[1], out[2])
